# baseline (speedup 1.0000x reference)
import functools

import jax
import jax.numpy as jnp
from jax import lax
from jax.experimental import pallas as pl
from jax.experimental.pallas import tpu as pltpu
from jax.experimental.pallas import tpu_sc as plsc

B = 4096
D = 50
DP = 128
NTAB = 13
NF = 2 * NTAB
NUM = 13
NW = 32
BPW = B // NW
BN_EPS = 1e-5

S_CHUNK, S_NC = 384, 3
S_ROWS = S_NC * S_CHUNK // 2
S_SENT = 504
B_CHUNK, B_NC = 3200, 32
B_ROWS = B_NC * B_CHUNK // 2
B_SENT = 50000


def _pack_tables(emb_t, vocab, chunk, nc, zero_from):
    half = chunk // 2

    def body(src_ref, o_ref):
        c = pl.program_id(1)
        x = src_ref[0].T.reshape(half, 2, D)
        o_ref[:, :D] = x[:, 0, :]
        o_ref[:, D:2 * D] = x[:, 1, :]
        o_ref[:, 2 * D:] = jnp.zeros((half, DP - 2 * D), jnp.float32)

        @pl.when(c == nc - 1)
        def _():
            o_ref[zero_from - (nc - 1) * half:, :] = jnp.zeros(
                (nc * half - zero_from, DP), jnp.float32)

    return pl.pallas_call(
        body,
        grid=(NTAB, nc),
        in_specs=[pl.BlockSpec((1, D, chunk), lambda t, c: (t, 0, c))],
        out_specs=pl.BlockSpec((half, DP), lambda t, c: (t * nc + c, 0)),
        out_shape=jax.ShapeDtypeStruct((NTAB * nc * half, DP), jnp.float32),
    )(emb_t)


def _sc_gather_body(small_hbm, big_hbm, idx_hbm, xa_hbm, xb_hbm, idxs_v,
                    rows0_v, rows1_v, sem0, sem1):
    c = lax.axis_index("c")
    s = lax.axis_index("s")
    wid = s * 2 + c
    base = wid * BPW
    rows = (rows0_v, rows1_v)
    sems = (sem0, sem1)
    pltpu.sync_copy(idx_hbm.at[:, pl.ds(base, BPW)], idxs_v)

    def start(step):
        f = step // 2
        tbl = small_hbm if f < NTAB else big_hbm
        pltpu.async_copy(tbl.at[idxs_v.at[step]], rows[step % 2],
                         sems[step % 2])

    start(0)
    for step in range(2 * NF):
        if step + 1 < 2 * NF:
            start(step + 1)
        pltpu.make_async_copy(
            small_hbm.at[pl.ds(0, BPW)], rows[step % 2],
            sems[step % 2]).wait()
        out = xa_hbm if step % 2 == 0 else xb_hbm
        pltpu.sync_copy(rows[step % 2],
                        out.at[step // 2, pl.ds(base, BPW), :])


@jax.jit
def _sc_gather(small_p, big_p, idx_ab):
    mesh = plsc.VectorSubcoreMesh(core_axis_name="c", subcore_axis_name="s")
    out = jax.ShapeDtypeStruct((NF, B, DP), jnp.float32)
    return pl.kernel(
        _sc_gather_body,
        out_type=(out, out),
        mesh=mesh,
        scratch_types=[
            pltpu.VMEM((2 * NF, BPW), jnp.int32),
            pltpu.VMEM((BPW, DP), jnp.float32),
            pltpu.VMEM((BPW, DP), jnp.float32),
            pltpu.SemaphoreType.DMA,
            pltpu.SemaphoreType.DMA,
        ],
    )(small_p, big_p, idx_ab)


def _mlp_body(xa_ref, xb_ref, num_ref, w1a_ref, w1b_ref, w1n_ref, b1_ref,
              w2_ref, b2_ref, w3_ref, b3_ref, w4_ref, b4_ref, o_ref):
    h = jnp.dot(num_ref[...], w1n_ref[...], preferred_element_type=jnp.float32)
    for f in range(NF):
        h += jnp.dot(xa_ref[f].astype(jnp.bfloat16), w1a_ref[f],
                     preferred_element_type=jnp.float32)
        h += jnp.dot(xb_ref[f].astype(jnp.bfloat16), w1b_ref[f],
                     preferred_element_type=jnp.float32)
    h = jnp.maximum(h + b1_ref[...], 0.0).astype(jnp.bfloat16)
    h = jnp.dot(h, w2_ref[...], preferred_element_type=jnp.float32)
    h = jnp.maximum(h + b2_ref[...], 0.0).astype(jnp.bfloat16)
    h = jnp.dot(h, w3_ref[...], preferred_element_type=jnp.float32)
    h = jnp.maximum(h + b3_ref[...], 0.0).astype(jnp.bfloat16)
    o_ref[...] = (jnp.dot(h, w4_ref[...], preferred_element_type=jnp.float32)
                  + b4_ref[...])


def _mlp(xa, xb, num, w1a, w1b, w1n, b1, w2t, b2, w3t, b3, w4t, b4):
    bb = 512
    grid = (B // bb,)
    full2 = lambda a: pl.BlockSpec(a.shape, lambda i: (0, 0))
    full3 = lambda a: pl.BlockSpec(a.shape, lambda i: (0, 0, 0))
    xspec = pl.BlockSpec((NF, bb, DP), lambda i: (0, i, 0))
    return pl.pallas_call(
        _mlp_body,
        grid=grid,
        in_specs=[
            xspec, xspec,
            pl.BlockSpec((bb, NUM), lambda i: (i, 0)),
            full3(w1a), full3(w1b), full2(w1n), full2(b1),
            full2(w2t), full2(b2), full2(w3t), full2(b3),
            full2(w4t), full2(b4),
        ],
        out_specs=pl.BlockSpec((bb, 1), lambda i: (i, 0)),
        out_shape=jax.ShapeDtypeStruct((B, 1), jnp.float32),
    )(xa, xb, num, w1a, w1b, w1n, b1, w2t, b2, w3t, b3, w4t, b4)


def kernel(cat_features, num_features, emb_small, emb_big,
           W1, b1, gamma, beta, W2, b2, W3, b3, W4, b4):
    small_p = _pack_tables(emb_small.transpose(0, 2, 1), 1000, S_CHUNK,
                           S_NC, S_SENT)
    big_p = _pack_tables(emb_big.transpose(0, 2, 1), 100000, B_CHUNK,
                         B_NC, B_SENT)

    cat_t = cat_features.T.astype(jnp.int32)
    tnum = jnp.arange(NTAB, dtype=jnp.int32)[:, None]
    bpos = jnp.arange(B, dtype=jnp.int32)[None, :]
    v_s, v_b = cat_t[:NTAB], cat_t[NTAB:]
    row_s = tnum * S_ROWS + v_s // 2
    row_b = tnum * B_ROWS + v_b // 2
    sent_s = tnum * S_ROWS + S_SENT + bpos % (S_ROWS - S_SENT)
    sent_b = tnum * B_ROWS + B_SENT + bpos % (B_ROWS - B_SENT)
    even_s, even_b = v_s % 2 == 0, v_b % 2 == 0
    idx_a = jnp.concatenate([jnp.where(even_s, row_s, sent_s),
                             jnp.where(even_b, row_b, sent_b)], axis=0)
    idx_b = jnp.concatenate([jnp.where(even_s, sent_s, row_s),
                             jnp.where(even_b, sent_b, row_b)], axis=0)
    idx_ab = jnp.stack([idx_a, idx_b], axis=1).reshape(2 * NF, B)

    scale = gamma / jnp.sqrt(1.0 + BN_EPS)
    w2f = (W2 * scale[None, :]).T.astype(jnp.bfloat16)
    b2f = b2 + W2 @ beta

    w1slab = W1[:, :NF * D].T.reshape(NF, D, 512).astype(jnp.bfloat16)
    w1a = jnp.pad(w1slab, [(0, 0), (0, DP - D), (0, 0)])
    w1b = jnp.pad(w1slab, [(0, 0), (D, DP - 2 * D), (0, 0)])
    w1n = W1[:, NF * D:].T.astype(jnp.bfloat16)

    xa, xb = _sc_gather(small_p, big_p, idx_ab)
    out = _mlp(xa, xb, num_features.astype(jnp.bfloat16), w1a, w1b, w1n,
               b1[None, :], w2f, b2f[None, :],
               W3.T.astype(jnp.bfloat16), b3[None, :],
               W4.T.astype(jnp.bfloat16), b4[None, :])
    return out[:, 0]


# speedup vs baseline: 1.7026x; 1.7026x over previous
import functools

import jax
import jax.numpy as jnp
from jax import lax
from jax.experimental import pallas as pl
from jax.experimental.pallas import tpu as pltpu
from jax.experimental.pallas import tpu_sc as plsc

B = 4096
D = 50
DP = 128
NTAB = 13
NF = 2 * NTAB
NUM = 13
NW = 32
BPW = B // NW
BN_EPS = 1e-5


def _sc_gather_body(small_hbm, big_hbm, idx_hbm, x_hbm, idxs_v,
                    rows0_v, rows1_v, sem0, sem1):
    c = lax.axis_index("c")
    s = lax.axis_index("s")
    wid = s * 2 + c
    base = wid * BPW
    rows = (rows0_v, rows1_v)
    sems = (sem0, sem1)
    pltpu.sync_copy(idx_hbm.at[:, pl.ds(base, BPW)], idxs_v)

    def start(f):
        tbl = small_hbm if f < NTAB else big_hbm
        pltpu.async_copy(tbl.at[idxs_v.at[f]], rows[f % 2], sems[f % 2])

    start(0)
    for f in range(NF):
        if f + 1 < NF:
            start(f + 1)
        pltpu.make_async_copy(
            small_hbm.at[pl.ds(0, BPW)], rows[f % 2], sems[f % 2]).wait()
        pltpu.sync_copy(rows[f % 2], x_hbm.at[f, pl.ds(base, BPW), :])


@jax.jit
def _sc_gather(small_p, big_p, idx2):
    mesh = plsc.VectorSubcoreMesh(core_axis_name="c", subcore_axis_name="s")
    return pl.kernel(
        _sc_gather_body,
        out_type=jax.ShapeDtypeStruct((NF, B, DP), jnp.float32),
        mesh=mesh,
        scratch_types=[
            pltpu.VMEM((NF, BPW), jnp.int32),
            pltpu.VMEM((BPW, DP), jnp.float32),
            pltpu.VMEM((BPW, DP), jnp.float32),
            pltpu.SemaphoreType.DMA,
            pltpu.SemaphoreType.DMA,
        ],
    )(small_p, big_p, idx2)


def _pad_tables(emb_t, vocab, chunk):
    nc = -(-vocab // chunk)

    def body(src_ref, o_ref):
        x = src_ref[0]
        o_ref[:, :D] = x.T
        o_ref[:, D:] = jnp.zeros((chunk, DP - D), jnp.float32)

    return pl.pallas_call(
        body,
        grid=(NTAB, nc),
        in_specs=[pl.BlockSpec((1, D, chunk), lambda t, c: (t, 0, c))],
        out_specs=pl.BlockSpec((chunk, DP), lambda t, c: (t * nc + c, 0)),
        out_shape=jax.ShapeDtypeStruct((NTAB * nc * chunk, DP), jnp.float32),
    )(emb_t)


def _mlp_body(x_ref, num_ref, w1_ref, w1n_ref, b1_ref, w2_ref, b2_ref,
              w3_ref, b3_ref, w4_ref, b4_ref, o_ref):
    h = jnp.dot(num_ref[...], w1n_ref[...], preferred_element_type=jnp.float32)
    for f in range(NF):
        h += jnp.dot(x_ref[f].astype(jnp.bfloat16), w1_ref[f],
                     preferred_element_type=jnp.float32)
    h = jnp.maximum(h + b1_ref[...], 0.0).astype(jnp.bfloat16)
    h = jnp.dot(h, w2_ref[...], preferred_element_type=jnp.float32)
    h = jnp.maximum(h + b2_ref[...], 0.0).astype(jnp.bfloat16)
    h = jnp.dot(h, w3_ref[...], preferred_element_type=jnp.float32)
    h = jnp.maximum(h + b3_ref[...], 0.0).astype(jnp.bfloat16)
    o_ref[...] = (jnp.dot(h, w4_ref[...], preferred_element_type=jnp.float32)
                  + b4_ref[...])


def _mlp(x3, num, w1p, w1n, b1, w2t, b2, w3t, b3, w4t, b4):
    bb = 512
    grid = (B // bb,)
    full2 = lambda a: pl.BlockSpec(a.shape, lambda i: (0, 0))
    full3 = lambda a: pl.BlockSpec(a.shape, lambda i: (0, 0, 0))
    return pl.pallas_call(
        _mlp_body,
        grid=grid,
        in_specs=[
            pl.BlockSpec((NF, bb, DP), lambda i: (0, i, 0)),
            pl.BlockSpec((bb, NUM), lambda i: (i, 0)),
            full3(w1p), full2(w1n), full2(b1),
            full2(w2t), full2(b2), full2(w3t), full2(b3),
            full2(w4t), full2(b4),
        ],
        out_specs=pl.BlockSpec((bb, 1), lambda i: (i, 0)),
        out_shape=jax.ShapeDtypeStruct((B, 1), jnp.float32),
    )(x3, num, w1p, w1n, b1, w2t, b2, w3t, b3, w4t, b4)


def kernel(cat_features, num_features, emb_small, emb_big,
           W1, b1, gamma, beta, W2, b2, W3, b3, W4, b4):
    small_p = _pad_tables(emb_small.transpose(0, 2, 1), 1000, 1000)
    big_p = _pad_tables(emb_big.transpose(0, 2, 1), 100000, 6400)

    offs_small = (jnp.arange(NTAB, dtype=jnp.int32) * 1000)[:, None]
    offs_big = (jnp.arange(NTAB, dtype=jnp.int32) * 102400)[:, None]
    cat_t = cat_features.T.astype(jnp.int32)
    idx2 = jnp.concatenate(
        [cat_t[:NTAB] + offs_small, cat_t[NTAB:] + offs_big], axis=0)

    scale = gamma / jnp.sqrt(1.0 + BN_EPS)
    w2f = (W2 * scale[None, :]).T.astype(jnp.bfloat16)
    b2f = b2 + W2 @ beta

    w1p = jnp.pad(W1[:, :NF * D].T.reshape(NF, D, 512).astype(jnp.bfloat16),
                  [(0, 0), (0, DP - D), (0, 0)])
    w1n = W1[:, NF * D:].T.astype(jnp.bfloat16)

    x3 = _sc_gather(small_p, big_p, idx2)
    out = _mlp(x3, num_features.astype(jnp.bfloat16), w1p, w1n,
               b1[None, :], w2f, b2f[None, :],
               W3.T.astype(jnp.bfloat16), b3[None, :],
               W4.T.astype(jnp.bfloat16), b4[None, :])
    return out[:, 0]


# speedup vs baseline: 1.9140x; 1.1241x over previous
import functools

import jax
import jax.numpy as jnp
from jax import lax
from jax.experimental import pallas as pl
from jax.experimental.pallas import tpu as pltpu
from jax.experimental.pallas import tpu_sc as plsc

B = 4096
D = 50
DP = 128
NTAB = 13
NF = 2 * NTAB
NUM = 13
NW = 32
BPW = B // NW
BN_EPS = 1e-5


def _sc_gather_body(small_hbm, big_hbm, idx_hbm, x_hbm, idxs_v,
                    rows0_v, rows1_v, sem0, sem1):
    c = lax.axis_index("c")
    s = lax.axis_index("s")
    wid = s * 2 + c
    base = wid * BPW
    rows = (rows0_v, rows1_v)
    sems = (sem0, sem1)
    pltpu.sync_copy(idx_hbm.at[:, pl.ds(base, BPW)], idxs_v)

    def start(f):
        tbl = small_hbm if f < NTAB else big_hbm
        pltpu.async_copy(tbl.at[idxs_v.at[f]], rows[f % 2], sems[f % 2])

    start(0)
    for f in range(NF):
        if f + 1 < NF:
            start(f + 1)
        pltpu.make_async_copy(
            small_hbm.at[pl.ds(0, BPW)], rows[f % 2], sems[f % 2]).wait()
        pltpu.sync_copy(rows[f % 2], x_hbm.at[f, pl.ds(base, BPW), :])


@jax.jit
def _sc_gather(small_p, big_p, idx2):
    mesh = plsc.VectorSubcoreMesh(core_axis_name="c", subcore_axis_name="s")
    return pl.kernel(
        _sc_gather_body,
        out_type=jax.ShapeDtypeStruct((NF, B, DP), jnp.float32),
        mesh=mesh,
        scratch_types=[
            pltpu.VMEM((NF, BPW), jnp.int32),
            pltpu.VMEM((BPW, DP), jnp.float32),
            pltpu.VMEM((BPW, DP), jnp.float32),
            pltpu.SemaphoreType.DMA,
            pltpu.SemaphoreType.DMA,
        ],
    )(small_p, big_p, idx2)


def _pad_tables(emb_t, vocab, chunk):
    nc = -(-vocab // chunk)

    def body(src_ref, o_ref):
        x = src_ref[0]
        o_ref[:, :D] = x.T
        o_ref[:, D:] = jnp.zeros((chunk, DP - D), jnp.float32)

    return pl.pallas_call(
        body,
        grid=(NTAB, nc),
        in_specs=[pl.BlockSpec((1, D, chunk), lambda t, c: (t, 0, c))],
        out_specs=pl.BlockSpec((chunk, DP), lambda t, c: (t * nc + c, 0)),
        out_shape=jax.ShapeDtypeStruct((NTAB * nc * chunk, DP), jnp.float32),
    )(emb_t)


def _mlp_body(x_ref, num_ref, w1_ref, w1n_ref, b1_ref, w2_ref, b2_ref,
              w3_ref, b3_ref, w4_ref, b4_ref, o_ref):
    h = jnp.dot(num_ref[...], w1n_ref[...], preferred_element_type=jnp.float32)
    for f in range(NF):
        h += jnp.dot(x_ref[f].astype(jnp.bfloat16), w1_ref[f],
                     preferred_element_type=jnp.float32)
    h = jnp.maximum(h + b1_ref[...], 0.0).astype(jnp.bfloat16)
    h = jnp.dot(h, w2_ref[...], preferred_element_type=jnp.float32)
    h = jnp.maximum(h + b2_ref[...], 0.0).astype(jnp.bfloat16)
    h = jnp.dot(h, w3_ref[...], preferred_element_type=jnp.float32)
    h = jnp.maximum(h + b3_ref[...], 0.0).astype(jnp.bfloat16)
    o_ref[...] = (jnp.dot(h, w4_ref[...], preferred_element_type=jnp.float32)
                  + b4_ref[...])


def _mlp(x3, num, w1p, w1n, b1, w2t, b2, w3t, b3, w4t, b4):
    bb = 512
    grid = (B // bb,)
    full2 = lambda a: pl.BlockSpec(a.shape, lambda i: (0, 0))
    full3 = lambda a: pl.BlockSpec(a.shape, lambda i: (0, 0, 0))
    return pl.pallas_call(
        _mlp_body,
        grid=grid,
        in_specs=[
            pl.BlockSpec((NF, bb, DP), lambda i: (0, i, 0)),
            pl.BlockSpec((bb, NUM), lambda i: (i, 0)),
            full3(w1p), full2(w1n), full2(b1),
            full2(w2t), full2(b2), full2(w3t), full2(b3),
            full2(w4t), full2(b4),
        ],
        out_specs=pl.BlockSpec((bb, 1), lambda i: (i, 0)),
        out_shape=jax.ShapeDtypeStruct((B, 1), jnp.float32),
    )(x3, num, w1p, w1n, b1, w2t, b2, w3t, b3, w4t, b4)


def kernel(cat_features, num_features, emb_small, emb_big,
           W1, b1, gamma, beta, W2, b2, W3, b3, W4, b4):
    small_p = _pad_tables(emb_small.transpose(0, 2, 1), 1000, 1000)
    big_p = _pad_tables(emb_big.transpose(0, 2, 1), 100000, 12800)

    offs_small = (jnp.arange(NTAB, dtype=jnp.int32) * 1000)[:, None]
    offs_big = (jnp.arange(NTAB, dtype=jnp.int32) * 102400)[:, None]
    cat_t = cat_features.T.astype(jnp.int32)
    idx2 = jnp.concatenate(
        [cat_t[:NTAB] + offs_small, cat_t[NTAB:] + offs_big], axis=0)

    scale = gamma / jnp.sqrt(1.0 + BN_EPS)
    w2f = (W2 * scale[None, :]).T.astype(jnp.bfloat16)
    b2f = b2 + W2 @ beta

    w1p = jnp.pad(W1[:, :NF * D].T.reshape(NF, D, 512).astype(jnp.bfloat16),
                  [(0, 0), (0, DP - D), (0, 0)])
    w1n = W1[:, NF * D:].T.astype(jnp.bfloat16)

    x3 = _sc_gather(small_p, big_p, idx2)
    out = _mlp(x3, num_features.astype(jnp.bfloat16), w1p, w1n,
               b1[None, :], w2f, b2f[None, :],
               W3.T.astype(jnp.bfloat16), b3[None, :],
               W4.T.astype(jnp.bfloat16), b4[None, :])
    return out[:, 0]


# speedup vs baseline: 1.9675x; 1.0280x over previous
import functools

import jax
import jax.numpy as jnp
from jax import lax
from jax.experimental import pallas as pl
from jax.experimental.pallas import tpu as pltpu
from jax.experimental.pallas import tpu_sc as plsc

B = 4096
D = 50
DP = 128
NTAB = 13
NF = 2 * NTAB
NUM = 13
NW = 32
BPW = B // NW
BN_EPS = 1e-5


def _sc_gather_body(small_hbm, big_hbm, idx_hbm, x_hbm, idxs_v,
                    rows0_v, rows1_v, sem0, sem1):
    c = lax.axis_index("c")
    s = lax.axis_index("s")
    wid = s * 2 + c
    base = wid * BPW
    rows = (rows0_v, rows1_v)
    sems = (sem0, sem1)
    pltpu.sync_copy(idx_hbm.at[:, pl.ds(base, BPW)], idxs_v)

    def start(f):
        tbl = small_hbm if f < NTAB else big_hbm
        pltpu.async_copy(tbl.at[idxs_v.at[f]], rows[f % 2], sems[f % 2])

    start(0)
    for f in range(NF):
        if f + 1 < NF:
            start(f + 1)
        pltpu.make_async_copy(
            small_hbm.at[pl.ds(0, BPW)], rows[f % 2], sems[f % 2]).wait()
        pltpu.sync_copy(rows[f % 2], x_hbm.at[f, pl.ds(base, BPW), :])


@jax.jit
def _sc_gather(small_p, big_p, idx2):
    mesh = plsc.VectorSubcoreMesh(core_axis_name="c", subcore_axis_name="s")
    return pl.kernel(
        _sc_gather_body,
        out_type=jax.ShapeDtypeStruct((NF, B, DP), jnp.float32),
        mesh=mesh,
        scratch_types=[
            pltpu.VMEM((NF, BPW), jnp.int32),
            pltpu.VMEM((BPW, DP), jnp.float32),
            pltpu.VMEM((BPW, DP), jnp.float32),
            pltpu.SemaphoreType.DMA,
            pltpu.SemaphoreType.DMA,
        ],
    )(small_p, big_p, idx2)


def _pad_tables(emb_t, vocab, chunk):
    nc = -(-vocab // chunk)

    def body(src_ref, o_ref):
        x = src_ref[0]
        o_ref[:, :D] = x.T
        o_ref[:, D:] = jnp.zeros((chunk, DP - D), jnp.float32)

    return pl.pallas_call(
        body,
        grid=(NTAB, nc),
        in_specs=[pl.BlockSpec((1, D, chunk), lambda t, c: (t, 0, c))],
        out_specs=pl.BlockSpec((chunk, DP), lambda t, c: (t * nc + c, 0)),
        out_shape=jax.ShapeDtypeStruct((NTAB * nc * chunk, DP), jnp.float32),
    )(emb_t)


def _mlp_body(x_ref, num_ref, w1_ref, w1n_ref, b1_ref, w2_ref, b2_ref,
              w3_ref, b3_ref, w4_ref, b4_ref, o_ref):
    h = jnp.dot(num_ref[...], w1n_ref[...], preferred_element_type=jnp.float32)
    for f in range(NF):
        h += jnp.dot(x_ref[f].astype(jnp.bfloat16), w1_ref[f],
                     preferred_element_type=jnp.float32)
    h = jnp.maximum(h + b1_ref[...], 0.0).astype(jnp.bfloat16)
    h = jnp.dot(h, w2_ref[...], preferred_element_type=jnp.float32)
    h = jnp.maximum(h + b2_ref[...], 0.0).astype(jnp.bfloat16)
    h = jnp.dot(h, w3_ref[...], preferred_element_type=jnp.float32)
    h = jnp.maximum(h + b3_ref[...], 0.0).astype(jnp.bfloat16)
    o_ref[...] = (jnp.dot(h, w4_ref[...], preferred_element_type=jnp.float32)
                  + b4_ref[...])


def _mlp(x3, num, w1p, w1n, b1, w2t, b2, w3t, b3, w4t, b4):
    bb = 512
    grid = (B // bb,)
    full2 = lambda a: pl.BlockSpec(a.shape, lambda i: (0, 0))
    full3 = lambda a: pl.BlockSpec(a.shape, lambda i: (0, 0, 0))
    return pl.pallas_call(
        _mlp_body,
        grid=grid,
        in_specs=[
            pl.BlockSpec((NF, bb, DP), lambda i: (0, i, 0)),
            pl.BlockSpec((bb, NUM), lambda i: (i, 0)),
            full3(w1p), full2(w1n), full2(b1),
            full2(w2t), full2(b2), full2(w3t), full2(b3),
            full2(w4t), full2(b4),
        ],
        out_specs=pl.BlockSpec((bb, 1), lambda i: (i, 0)),
        out_shape=jax.ShapeDtypeStruct((B, 1), jnp.float32),
    )(x3, num, w1p, w1n, b1, w2t, b2, w3t, b3, w4t, b4)


def kernel(cat_features, num_features, emb_small, emb_big,
           W1, b1, gamma, beta, W2, b2, W3, b3, W4, b4):
    small_p = _pad_tables(emb_small.transpose(0, 2, 1), 1000, 1000)
    big_p = _pad_tables(emb_big.transpose(0, 2, 1), 100000, 25600)

    offs_small = (jnp.arange(NTAB, dtype=jnp.int32) * 1000)[:, None]
    offs_big = (jnp.arange(NTAB, dtype=jnp.int32) * 102400)[:, None]
    cat_t = cat_features.T.astype(jnp.int32)
    idx2 = jnp.concatenate(
        [cat_t[:NTAB] + offs_small, cat_t[NTAB:] + offs_big], axis=0)

    scale = gamma / jnp.sqrt(1.0 + BN_EPS)
    w2f = (W2 * scale[None, :]).T.astype(jnp.bfloat16)
    b2f = b2 + W2 @ beta

    w1p = jnp.pad(W1[:, :NF * D].T.reshape(NF, D, 512).astype(jnp.bfloat16),
                  [(0, 0), (0, DP - D), (0, 0)])
    w1n = W1[:, NF * D:].T.astype(jnp.bfloat16)

    x3 = _sc_gather(small_p, big_p, idx2)
    out = _mlp(x3, num_features.astype(jnp.bfloat16), w1p, w1n,
               b1[None, :], w2f, b2f[None, :],
               W3.T.astype(jnp.bfloat16), b3[None, :],
               W4.T.astype(jnp.bfloat16), b4[None, :])
    return out[:, 0]
